# baseline (device time: 46128 ns/iter reference)
import jax
import jax.numpy as jnp
from jax import lax
from jax.experimental import pallas as pl
from jax.experimental.pallas import tpu as pltpu

N_DEV = 32
T_LOC = 16
D = 512
E_LOC = 2
E = N_DEV * E_LOC
T = N_DEV * T_LOC
TILE = 256


def kernel(x, router, W1, W2):
    rt = router.T
    W1b = W1.astype(jnp.bfloat16)
    W2b = W2.astype(jnp.bfloat16)

    def body(x_ref, rt_ref, w1_ref, w2_ref, out_ref,
             xg_ref, rg_ref, part_ref, pbuf_ref,
             xsend, xrecv, rsend, rrecv, psend, precv):
        my_d = lax.axis_index("i")

        bsem = pltpu.get_barrier_semaphore()
        for j in range(1, N_DEV):
            p = lax.rem(my_d + j, N_DEV)
            pl.semaphore_signal(bsem, inc=1, device_id=(p,),
                                device_id_type=pl.DeviceIdType.MESH)
        pl.semaphore_wait(bsem, N_DEV - 1)

        xg_ref[pl.ds(my_d, 1)] = x_ref[...].reshape(1, T_LOC, D)
        rg_ref[pl.ds(my_d, 1)] = rt_ref[...].reshape(1, E_LOC, D)

        sends = []
        for j in range(1, N_DEV):
            p = lax.rem(my_d + j, N_DEV)
            c = pltpu.make_async_remote_copy(
                src_ref=x_ref, dst_ref=xg_ref.at[my_d],
                send_sem=xsend.at[j], recv_sem=xrecv.at[my_d],
                device_id=(p,), device_id_type=pl.DeviceIdType.MESH)
            c.start()
            sends.append(c)
            c = pltpu.make_async_remote_copy(
                src_ref=rt_ref, dst_ref=rg_ref.at[my_d],
                send_sem=rsend.at[j], recv_sem=rrecv.at[my_d],
                device_id=(p,), device_id_type=pl.DeviceIdType.MESH)
            c.start()
            sends.append(c)

        for j in range(1, N_DEV):
            s = lax.rem(my_d + j, N_DEV)
            pltpu.make_async_remote_copy(
                src_ref=x_ref, dst_ref=xg_ref.at[s],
                send_sem=xsend.at[0], recv_sem=xrecv.at[s],
                device_id=(my_d,), device_id_type=pl.DeviceIdType.MESH,
            ).wait_recv()
            pltpu.make_async_remote_copy(
                src_ref=rt_ref, dst_ref=rg_ref.at[s],
                send_sem=rsend.at[0], recv_sem=rrecv.at[s],
                device_id=(my_d,), device_id_type=pl.DeviceIdType.MESH,
            ).wait_recv()

        xf = xg_ref[...].reshape(T, D)
        rT = rg_ref[...].reshape(E, D)
        gates = lax.dot_general(
            xf, rT, (((1,), (1,)), ((), ())),
            precision=lax.Precision.HIGHEST,
            preferred_element_type=jnp.float32)
        iota = lax.broadcasted_iota(jnp.int32, (T, E), 1)
        m1 = jnp.max(gates, axis=1, keepdims=True)
        a1 = jnp.min(jnp.where(gates >= m1, iota, E), axis=1, keepdims=True)
        g2 = jnp.where(iota == a1, -1e30, gates)
        m2 = jnp.max(g2, axis=1, keepdims=True)
        a2 = jnp.min(jnp.where(g2 >= m2, iota, E), axis=1, keepdims=True)
        w1 = 1.0 / (1.0 + jnp.exp(m2 - m1))
        w2 = 1.0 - w1
        e0 = 2 * my_d
        wv0 = jnp.where(a1 == e0, w1, 0.0) + jnp.where(a2 == e0, w2, 0.0)
        wv1 = jnp.where(a1 == e0 + 1, w1, 0.0) + jnp.where(a2 == e0 + 1, w2, 0.0)

        xb = xf.astype(jnp.bfloat16)
        for ti in range(T // TILE):
            xt = xb[ti * TILE:(ti + 1) * TILE]
            acc = jnp.zeros((TILE, D), jnp.float32)
            for le, wv in ((0, wv0), (1, wv1)):
                h = lax.dot_general(
                    xt, w1_ref[le], (((1,), (0,)), ((), ())),
                    preferred_element_type=jnp.float32)
                h = jnp.maximum(h, 0.0).astype(jnp.bfloat16)
                y = lax.dot_general(
                    h, w2_ref[le], (((1,), (0,)), ((), ())),
                    preferred_element_type=jnp.float32)
                acc = acc + y * wv[ti * TILE:(ti + 1) * TILE]
            nb = TILE // T_LOC
            part_ref[pl.ds(ti * nb, nb)] = acc.reshape(nb, T_LOC, D)

        pbuf_ref[pl.ds(my_d, 1)] = part_ref[pl.ds(my_d, 1)]
        for j in range(1, N_DEV):
            p = lax.rem(my_d + j, N_DEV)
            c = pltpu.make_async_remote_copy(
                src_ref=part_ref.at[p], dst_ref=pbuf_ref.at[my_d],
                send_sem=psend.at[j], recv_sem=precv.at[my_d],
                device_id=(p,), device_id_type=pl.DeviceIdType.MESH)
            c.start()
            sends.append(c)
        for j in range(1, N_DEV):
            s = lax.rem(my_d + j, N_DEV)
            pltpu.make_async_remote_copy(
                src_ref=part_ref.at[0], dst_ref=pbuf_ref.at[s],
                send_sem=psend.at[0], recv_sem=precv.at[s],
                device_id=(my_d,), device_id_type=pl.DeviceIdType.MESH,
            ).wait_recv()

        out_ref[...] = jnp.sum(pbuf_ref[...], axis=0)

        for c in sends:
            c.wait_send()

    return pl.pallas_call(
        body,
        out_shape=jax.ShapeDtypeStruct((T_LOC, D), jnp.float32),
        in_specs=[pl.BlockSpec(memory_space=pltpu.VMEM)] * 4,
        out_specs=pl.BlockSpec(memory_space=pltpu.VMEM),
        scratch_shapes=[
            pltpu.VMEM((N_DEV, T_LOC, D), jnp.float32),
            pltpu.VMEM((N_DEV, E_LOC, D), jnp.float32),
            pltpu.VMEM((N_DEV, T_LOC, D), jnp.float32),
            pltpu.VMEM((N_DEV, T_LOC, D), jnp.float32),
            pltpu.SemaphoreType.DMA((N_DEV,)),
            pltpu.SemaphoreType.DMA((N_DEV,)),
            pltpu.SemaphoreType.DMA((N_DEV,)),
            pltpu.SemaphoreType.DMA((N_DEV,)),
            pltpu.SemaphoreType.DMA((N_DEV,)),
            pltpu.SemaphoreType.DMA((N_DEV,)),
        ],
        compiler_params=pltpu.CompilerParams(collective_id=0),
    )(x, rt, W1b, W2b)


# device time: 43333 ns/iter; 1.0645x vs baseline; 1.0645x over previous
import jax
import jax.numpy as jnp
from jax import lax
from jax.experimental import pallas as pl
from jax.experimental.pallas import tpu as pltpu

N_DEV = 32
T_LOC = 16
D = 512
E_LOC = 2
E = N_DEV * E_LOC
T = N_DEV * T_LOC
NB = 4
SLOTS_PB = N_DEV // NB
ROWS_PB = SLOTS_PB * T_LOC


def kernel(x, router, W1, W2):
    rt = router.T
    W1b = W1.astype(jnp.bfloat16)
    W2b = W2.astype(jnp.bfloat16)

    def body(x_ref, rt_ref, w1_ref, w2_ref, out_ref,
             xg_ref, rg_ref, part_ref, pbuf_ref,
             xsend, xrecv, rsend, rrecv, psend, precv):
        my_d = lax.axis_index("i")
        mesh = pl.DeviceIdType.MESH

        bsem = pltpu.get_barrier_semaphore()
        for j in range(1, N_DEV):
            p = lax.rem(my_d + j, N_DEV)
            pl.semaphore_signal(bsem, inc=1, device_id=(p,),
                                device_id_type=mesh)
        pl.semaphore_wait(bsem, N_DEV - 1)

        sends = []
        for j in range(1, N_DEV):
            p = lax.rem(my_d + j, N_DEV)
            c = pltpu.make_async_remote_copy(
                src_ref=rt_ref, dst_ref=rg_ref.at[j - 1],
                send_sem=rsend.at[j - 1], recv_sem=rrecv.at[j - 1],
                device_id=(p,), device_id_type=mesh)
            c.start()
            sends.append(c)
        for j in range(1, N_DEV):
            p = lax.rem(my_d + j, N_DEV)
            c = pltpu.make_async_remote_copy(
                src_ref=x_ref, dst_ref=xg_ref.at[j - 1],
                send_sem=xsend.at[j - 1], recv_sem=xrecv.at[j - 1],
                device_id=(p,), device_id_type=mesh)
            c.start()
            sends.append(c)

        xg_ref[N_DEV - 1] = x_ref[...]
        rg_ref[N_DEV - 1] = rt_ref[...]

        for q in range(N_DEV - 1):
            pltpu.make_async_remote_copy(
                src_ref=rt_ref, dst_ref=rg_ref.at[q],
                send_sem=rsend.at[q], recv_sem=rrecv.at[q],
                device_id=(my_d,), device_id_type=mesh).wait_recv()
        rT = rg_ref[...].reshape(E, D)

        for b in range(NB):
            s0 = b * SLOTS_PB
            for q in range(s0, s0 + SLOTS_PB):
                if q == N_DEV - 1:
                    continue
                pltpu.make_async_remote_copy(
                    src_ref=x_ref, dst_ref=xg_ref.at[q],
                    send_sem=xsend.at[q], recv_sem=xrecv.at[q],
                    device_id=(my_d,), device_id_type=mesh).wait_recv()

            xf = xg_ref[s0:s0 + SLOTS_PB].reshape(ROWS_PB, D)
            gates = lax.dot_general(
                xf, rT, (((1,), (1,)), ((), ())),
                precision=lax.Precision.HIGHEST,
                preferred_element_type=jnp.float32)
            iota = lax.broadcasted_iota(jnp.int32, (ROWS_PB, E), 1)
            m1 = jnp.max(gates, axis=1, keepdims=True)
            a1 = jnp.min(jnp.where(gates >= m1, iota, E), axis=1,
                         keepdims=True)
            g2 = jnp.where(iota == a1, -1e30, gates)
            m2 = jnp.max(g2, axis=1, keepdims=True)
            a2 = jnp.min(jnp.where(g2 >= m2, iota, E), axis=1,
                         keepdims=True)
            w1 = 1.0 / (1.0 + jnp.exp(m2 - m1))
            w2 = 1.0 - w1
            wv0 = (jnp.where(a1 == E - 2, w1, 0.0)
                   + jnp.where(a2 == E - 2, w2, 0.0))
            wv1 = (jnp.where(a1 == E - 1, w1, 0.0)
                   + jnp.where(a2 == E - 1, w2, 0.0))

            xt = xf.astype(jnp.bfloat16)
            acc = jnp.zeros((ROWS_PB, D), jnp.float32)
            for le, wv in ((0, wv0), (1, wv1)):
                h = lax.dot_general(
                    xt, w1_ref[le], (((1,), (0,)), ((), ())),
                    preferred_element_type=jnp.float32)
                h = jnp.maximum(h, 0.0).astype(jnp.bfloat16)
                y = lax.dot_general(
                    h, w2_ref[le], (((1,), (0,)), ((), ())),
                    preferred_element_type=jnp.float32)
                acc = acc + y * wv
            part_ref[s0:s0 + SLOTS_PB] = acc.reshape(SLOTS_PB, T_LOC, D)

            for q in range(s0, s0 + SLOTS_PB):
                if q == N_DEV - 1:
                    pbuf_ref[N_DEV - 1] = part_ref[N_DEV - 1]
                    continue
                o = lax.rem(my_d + (N_DEV - 1 - q), N_DEV)
                c = pltpu.make_async_remote_copy(
                    src_ref=part_ref.at[q], dst_ref=pbuf_ref.at[q],
                    send_sem=psend.at[q], recv_sem=precv.at[q],
                    device_id=(o,), device_id_type=mesh)
                c.start()
                sends.append(c)

        for q in range(N_DEV - 1):
            pltpu.make_async_remote_copy(
                src_ref=part_ref.at[q], dst_ref=pbuf_ref.at[q],
                send_sem=psend.at[q], recv_sem=precv.at[q],
                device_id=(my_d,), device_id_type=mesh).wait_recv()
        out_ref[...] = jnp.sum(pbuf_ref[...], axis=0)

        for c in sends:
            c.wait_send()

    return pl.pallas_call(
        body,
        out_shape=jax.ShapeDtypeStruct((T_LOC, D), jnp.float32),
        in_specs=[pl.BlockSpec(memory_space=pltpu.VMEM)] * 4,
        out_specs=pl.BlockSpec(memory_space=pltpu.VMEM),
        scratch_shapes=[
            pltpu.VMEM((N_DEV, T_LOC, D), jnp.float32),
            pltpu.VMEM((N_DEV, E_LOC, D), jnp.float32),
            pltpu.VMEM((N_DEV, T_LOC, D), jnp.float32),
            pltpu.VMEM((N_DEV, T_LOC, D), jnp.float32),
            pltpu.SemaphoreType.DMA((N_DEV,)),
            pltpu.SemaphoreType.DMA((N_DEV,)),
            pltpu.SemaphoreType.DMA((N_DEV,)),
            pltpu.SemaphoreType.DMA((N_DEV,)),
            pltpu.SemaphoreType.DMA((N_DEV,)),
            pltpu.SemaphoreType.DMA((N_DEV,)),
        ],
        compiler_params=pltpu.CompilerParams(collective_id=0),
    )(x, rt, W1b, W2b)
